# baseline (speedup 1.0000x reference)
import functools

import jax
import jax.numpy as jnp
from jax import lax
from jax.experimental import pallas as pl
from jax.experimental.pallas import tpu as pltpu
from jax.experimental.pallas import tpu_sc as plsc


ALPHA = 33
C = ALPHA + 1
NC = 2
NW = 32
ROWS_PER_W = 4
TCH = 1024
NCH = 2048 // TCH


def _sc_body(tok_hbm, eng_hbm, out_hbm, tok_v, eng_v, out_v):
    wid = lax.axis_index("s") * NC + lax.axis_index("c")
    for rb in range(ROWS_PER_W):
        b = wid * ROWS_PER_W + rb
        for tch in range(NCH):
            t0 = tch * TCH
            pltpu.sync_copy(tok_hbm.at[b, pl.ds(t0, TCH)], tok_v)
            pltpu.sync_copy(eng_hbm.at[b, pl.ds(t0, TCH)], eng_v)

            def chunk_body(i, _):
                v = tok_v[pl.ds(i * 16, 16)]
                e = eng_v[pl.ds(i * 16, 16)]
                for c in range(ALPHA):
                    out_v[c, pl.ds(i * 16, 16)] = jnp.where(
                        v == c, 1.0, 0.0
                    ).astype(jnp.float32)
                out_v[ALPHA, pl.ds(i * 16, 16)] = jnp.where(
                    e <= -1.0, 1.0, 0.0
                ).astype(jnp.float32)
                return 0

            lax.fori_loop(0, TCH // 16, chunk_body, 0)
            for c in range(C):
                pltpu.sync_copy(out_v.at[c], out_hbm.at[c, b, pl.ds(t0, TCH)])


@jax.jit
def _run_sc(tokens, energy_scores):
    nb, nt = tokens.shape
    mesh = plsc.VectorSubcoreMesh(core_axis_name="c", subcore_axis_name="s")
    k = functools.partial(
        pl.kernel,
        mesh=mesh,
        out_type=jax.ShapeDtypeStruct((C, nb, nt), jnp.float32),
        scratch_types=[
            pltpu.VMEM((TCH,), jnp.int32),
            pltpu.VMEM((TCH,), jnp.float32),
            pltpu.VMEM((C, TCH), jnp.float32),
        ],
    )(_sc_body)
    outp = k(tokens, energy_scores)
    return jnp.transpose(outp, (1, 2, 0))


def kernel(tokens, energy_scores):
    return _run_sc(tokens, energy_scores)


# speedup vs baseline: 5.9331x; 5.9331x over previous
import functools

import jax
import jax.numpy as jnp
from jax.experimental import pallas as pl


ALPHA = 33
C = ALPHA + 1


def _planes_body(tok_ref, eng_ref, out_ref):
    tok = tok_ref[...]
    for c in range(ALPHA):
        out_ref[c] = (tok == c).astype(jnp.float32)
    out_ref[ALPHA] = (eng_ref[...] <= -1.0).astype(jnp.float32)


@functools.partial(jax.jit, static_argnames=("block_rows",))
def _run(tokens, energy_scores, block_rows=16):
    nb, nt = tokens.shape
    outp = pl.pallas_call(
        _planes_body,
        grid=(nb // block_rows,),
        in_specs=[
            pl.BlockSpec((block_rows, nt), lambda i: (i, 0)),
            pl.BlockSpec((block_rows, nt), lambda i: (i, 0)),
        ],
        out_specs=pl.BlockSpec((C, block_rows, nt), lambda i: (0, i, 0)),
        out_shape=jax.ShapeDtypeStruct((C, nb, nt), jnp.float32),
    )(tokens, energy_scores)
    return jnp.transpose(outp, (1, 2, 0))


def kernel(tokens, energy_scores):
    return _run(tokens, energy_scores)
